# baseline (device time: 193849 ns/iter reference)
import jax
import jax.numpy as jnp
from jax import lax
from jax.experimental import pallas as pl
from jax.experimental.pallas import tpu as pltpu

NZ = 2
T = 1024
D = 1024
F = 4096
EL = 8
E = 16
C = 320
F_TILE = 1024

_MESH = pl.DeviceIdType.MESH


def _exchange_body(x_ref, r_ref, x_all_ref, idx_ref, w_ref,
                   rbuf, gbuf, send_sems, recv_sems):
    mx = lax.axis_index("x")
    my = lax.axis_index("y")
    mz = lax.axis_index("z")
    peer = (mx, my, 1 - mz)

    barrier = pltpu.get_barrier_semaphore()
    pl.semaphore_signal(barrier, inc=1, device_id=peer, device_id_type=_MESH)
    pl.semaphore_wait(barrier, 1)

    x_all_ref[mz] = x_ref[...].astype(jnp.bfloat16)
    rdma_x = pltpu.make_async_remote_copy(
        src_ref=x_all_ref.at[mz], dst_ref=x_all_ref.at[mz],
        send_sem=send_sems.at[0], recv_sem=recv_sems.at[0],
        device_id=peer, device_id_type=_MESH)
    rdma_x.start()

    rbuf[mz] = r_ref[...]
    rdma_r = pltpu.make_async_remote_copy(
        src_ref=rbuf.at[mz], dst_ref=rbuf.at[mz],
        send_sem=send_sems.at[1], recv_sem=recv_sems.at[1],
        device_id=peer, device_id_type=_MESH)
    rdma_r.start()
    rdma_r.wait()

    x = x_ref[...]
    gbuf[:, 0:EL] = jnp.dot(x, rbuf[0], preferred_element_type=jnp.float32,
                            precision=lax.Precision.HIGHEST)
    gbuf[:, EL:E] = jnp.dot(x, rbuf[1], preferred_element_type=jnp.float32,
                            precision=lax.Precision.HIGHEST)
    gates = gbuf[...]
    a1 = jnp.argmax(gates, axis=1)
    m1 = jnp.max(gates, axis=1)
    hit1 = lax.broadcasted_iota(jnp.int32, (T, E), 1) == a1[:, None]
    g2 = jnp.where(hit1, -1e30, gates)
    a2 = jnp.argmax(g2, axis=1)
    m2 = jnp.max(g2, axis=1)
    e2 = jnp.exp(m2 - m1)
    idx_ref[mz, 0] = a1.astype(jnp.int32)
    idx_ref[mz, 1] = a2.astype(jnp.int32)
    w_ref[mz, 0] = 1.0 / (1.0 + e2)
    w_ref[mz, 1] = e2 / (1.0 + e2)

    rdma_i = pltpu.make_async_remote_copy(
        src_ref=idx_ref.at[mz], dst_ref=idx_ref.at[mz],
        send_sem=send_sems.at[2], recv_sem=recv_sems.at[2],
        device_id=peer, device_id_type=_MESH)
    rdma_i.start()
    rdma_w = pltpu.make_async_remote_copy(
        src_ref=w_ref.at[mz], dst_ref=w_ref.at[mz],
        send_sem=send_sems.at[3], recv_sem=recv_sems.at[3],
        device_id=peer, device_id_type=_MESH)
    rdma_w.start()
    rdma_i.wait()
    rdma_w.wait()
    rdma_x.wait()


def _ffn_body(x_all_ref, tok_ref, w1_ref, w2_ref, wg_ref, y_ref, xg_s):
    ft = pl.program_id(1)

    @pl.when(ft == 0)
    def _():
        tok = tok_ref[0, 0]
        onehot = (lax.broadcasted_iota(jnp.int32, (C, NZ * T), 1)
                  == tok[:, None]).astype(jnp.bfloat16)
        xg_s[...] = jnp.dot(onehot, x_all_ref[...],
                            preferred_element_type=jnp.float32)

    h = jnp.dot(xg_s[...], w1_ref[0], preferred_element_type=jnp.float32)
    h = jnp.maximum(h, 0.0)
    y = jnp.dot(h, w2_ref[0], preferred_element_type=jnp.float32)
    y = y * wg_ref[0, 0][:, None]

    @pl.when(ft == 0)
    def _():
        y_ref[0] = y

    @pl.when(ft != 0)
    def _():
        y_ref[0] += y


def _combine_body(yg_ref, tok_ref, out_ref, sbuf, rbuf, send_sem, recv_sem):
    mx = lax.axis_index("x")
    my = lax.axis_index("y")
    mz = lax.axis_index("z")
    peer = (mx, my, 1 - mz)

    barrier = pltpu.get_barrier_semaphore()
    pl.semaphore_signal(barrier, inc=1, device_id=peer, device_id_type=_MESH)
    pl.semaphore_wait(barrier, 1)

    def half(base):
        acc = None
        for e in range(EL):
            tok = tok_ref[e]
            pt = (lax.broadcasted_iota(jnp.int32, (T, C), 0) + base
                  == tok[None, :]).astype(jnp.float32)
            c = jnp.dot(pt, yg_ref[e], preferred_element_type=jnp.float32)
            acc = c if acc is None else acc + c
        return acc

    sbuf[...] = half((1 - mz) * T).astype(jnp.bfloat16)
    rdma = pltpu.make_async_remote_copy(
        src_ref=sbuf, dst_ref=rbuf,
        send_sem=send_sem, recv_sem=recv_sem,
        device_id=peer, device_id_type=_MESH)
    rdma.start()
    mine = half(mz * T)
    rdma.wait()
    out_ref[...] = mine + rbuf[...].astype(jnp.float32)


def kernel(x, router, W1, W2):
    mz = lax.axis_index("z")

    x_all3, idx_all, w_all = pl.pallas_call(
        _exchange_body,
        out_shape=[
            jax.ShapeDtypeStruct((NZ, T, D), jnp.bfloat16),
            jax.ShapeDtypeStruct((NZ, 2, T), jnp.int32),
            jax.ShapeDtypeStruct((NZ, 2, T), jnp.float32),
        ],
        in_specs=[
            pl.BlockSpec(memory_space=pltpu.VMEM),
            pl.BlockSpec(memory_space=pltpu.VMEM),
        ],
        out_specs=[
            pl.BlockSpec(memory_space=pltpu.VMEM),
            pl.BlockSpec(memory_space=pltpu.VMEM),
            pl.BlockSpec(memory_space=pltpu.VMEM),
        ],
        scratch_shapes=[
            pltpu.VMEM((NZ, D, EL), jnp.float32),
            pltpu.VMEM((T, E), jnp.float32),
            pltpu.SemaphoreType.DMA((4,)),
            pltpu.SemaphoreType.DMA((4,)),
        ],
        compiler_params=pltpu.CompilerParams(collective_id=0),
    )(x, router)

    x_all = x_all3.reshape(NZ * T, D)
    le = idx_all - mz * EL
    le = jnp.where((le >= 0) & (le < EL), le, EL)
    le_f = le.reshape(-1)
    tok_f = jnp.broadcast_to(
        jnp.arange(NZ, dtype=jnp.int32)[:, None, None] * T
        + jnp.arange(T, dtype=jnp.int32)[None, None, :],
        (NZ, 2, T),
    ).reshape(-1)
    w_f = w_all.reshape(-1)

    le_s, tok_s, w_s = lax.sort((le_f, tok_f, w_f), num_keys=1)
    counts = jnp.sum(
        le_f[None, :] == jnp.arange(EL, dtype=jnp.int32)[:, None], axis=1)
    starts = jnp.concatenate(
        [jnp.zeros((1,), counts.dtype), jnp.cumsum(counts)[:-1]])
    tok_pad = jnp.concatenate([tok_s, jnp.zeros((C,), tok_s.dtype)])
    w_pad = jnp.concatenate([w_s, jnp.zeros((C,), w_s.dtype)])
    tok_g = jnp.stack(
        [lax.dynamic_slice(tok_pad, (starts[e],), (C,)) for e in range(EL)])
    w_g = jnp.stack(
        [lax.dynamic_slice(w_pad, (starts[e],), (C,)) for e in range(EL)])
    valid = jnp.arange(C, dtype=jnp.int32)[None, :] < counts[:, None]
    tok_g = jnp.where(valid, tok_g, 0).reshape(EL, 1, C)
    wg = jnp.where(valid, w_g, 0.0).reshape(EL, 1, C)

    yg = pl.pallas_call(
        _ffn_body,
        grid=(EL, F // F_TILE),
        in_specs=[
            pl.BlockSpec((NZ * T, D), lambda e, f: (0, 0)),
            pl.BlockSpec((1, 1, C), lambda e, f: (e, 0, 0)),
            pl.BlockSpec((1, D, F_TILE), lambda e, f: (e, 0, f)),
            pl.BlockSpec((1, F_TILE, D), lambda e, f: (e, f, 0)),
            pl.BlockSpec((1, 1, C), lambda e, f: (e, 0, 0)),
        ],
        out_specs=pl.BlockSpec((1, C, D), lambda e, f: (e, 0, 0)),
        out_shape=jax.ShapeDtypeStruct((EL, C, D), jnp.float32),
        scratch_shapes=[pltpu.VMEM((C, D), jnp.float32)],
        compiler_params=pltpu.CompilerParams(
            dimension_semantics=("arbitrary", "arbitrary")),
    )(x_all, tok_g, W1, W2, wg)

    out = pl.pallas_call(
        _combine_body,
        out_shape=jax.ShapeDtypeStruct((T, D), jnp.float32),
        in_specs=[
            pl.BlockSpec(memory_space=pltpu.VMEM),
            pl.BlockSpec(memory_space=pltpu.VMEM),
        ],
        out_specs=pl.BlockSpec(memory_space=pltpu.VMEM),
        scratch_shapes=[
            pltpu.VMEM((T, D), jnp.bfloat16),
            pltpu.VMEM((T, D), jnp.bfloat16),
            pltpu.SemaphoreType.DMA,
            pltpu.SemaphoreType.DMA,
        ],
        compiler_params=pltpu.CompilerParams(collective_id=1),
    )(yg, tok_g.reshape(EL, C))
    return out


# device time: 177072 ns/iter; 1.0947x vs baseline; 1.0947x over previous
import jax
import jax.numpy as jnp
from jax import lax
from jax.experimental import pallas as pl
from jax.experimental.pallas import tpu as pltpu

NZ = 2
T = 1024
D = 1024
F = 4096
EL = 8
E = 16
C = 320
F_TILE = 1024

_MESH = pl.DeviceIdType.MESH


def _exchange_body(x_ref, r_ref, x_all_ref, idx_ref, w_ref,
                   rbuf, gbuf, send_sems, recv_sems):
    mx = lax.axis_index("x")
    my = lax.axis_index("y")
    mz = lax.axis_index("z")
    peer = (mx, my, 1 - mz)

    barrier = pltpu.get_barrier_semaphore()
    pl.semaphore_signal(barrier, inc=1, device_id=peer, device_id_type=_MESH)
    pl.semaphore_wait(barrier, 1)

    rbuf[mz] = r_ref[...]
    rdma_r = pltpu.make_async_remote_copy(
        src_ref=rbuf.at[mz], dst_ref=rbuf.at[mz],
        send_sem=send_sems.at[1], recv_sem=recv_sems.at[1],
        device_id=peer, device_id_type=_MESH)
    rdma_r.start()

    x_all_ref[mz] = x_ref[...].astype(jnp.bfloat16)
    rdma_x = pltpu.make_async_remote_copy(
        src_ref=x_all_ref.at[mz], dst_ref=x_all_ref.at[mz],
        send_sem=send_sems.at[0], recv_sem=recv_sems.at[0],
        device_id=peer, device_id_type=_MESH)
    rdma_x.start()
    rdma_r.wait()

    x = x_ref[...]
    gbuf[:, 0:EL] = jnp.dot(x, rbuf[0], preferred_element_type=jnp.float32,
                            precision=lax.Precision.HIGHEST)
    gbuf[:, EL:E] = jnp.dot(x, rbuf[1], preferred_element_type=jnp.float32,
                            precision=lax.Precision.HIGHEST)
    gates = gbuf[...]
    a1 = jnp.argmax(gates, axis=1)
    m1 = jnp.max(gates, axis=1)
    hit1 = lax.broadcasted_iota(jnp.int32, (T, E), 1) == a1[:, None]
    g2 = jnp.where(hit1, -1e30, gates)
    a2 = jnp.argmax(g2, axis=1)
    m2 = jnp.max(g2, axis=1)
    e2 = jnp.exp(m2 - m1)
    idx_ref[mz, 0] = a1.astype(jnp.int32)
    idx_ref[mz, 1] = a2.astype(jnp.int32)
    w_ref[mz, 0] = 1.0 / (1.0 + e2)
    w_ref[mz, 1] = e2 / (1.0 + e2)

    rdma_i = pltpu.make_async_remote_copy(
        src_ref=idx_ref.at[mz], dst_ref=idx_ref.at[mz],
        send_sem=send_sems.at[2], recv_sem=recv_sems.at[2],
        device_id=peer, device_id_type=_MESH)
    rdma_i.start()
    rdma_w = pltpu.make_async_remote_copy(
        src_ref=w_ref.at[mz], dst_ref=w_ref.at[mz],
        send_sem=send_sems.at[3], recv_sem=recv_sems.at[3],
        device_id=peer, device_id_type=_MESH)
    rdma_w.start()
    rdma_i.wait()
    rdma_w.wait()
    rdma_x.wait()


def _ffn_body(x_all_ref, tok_ref, w1_ref, w2_ref, wg_ref, y_ref, xg_s):
    ft = pl.program_id(1)

    @pl.when(ft == 0)
    def _():
        tok = tok_ref[0, 0]
        onehot = (lax.broadcasted_iota(jnp.int32, (C, NZ * T), 1)
                  == tok[:, None]).astype(jnp.bfloat16)
        xg_s[...] = jnp.dot(onehot, x_all_ref[...],
                            preferred_element_type=jnp.float32)

    h = jnp.dot(xg_s[...], w1_ref[0], preferred_element_type=jnp.float32)
    h = jnp.maximum(h, 0.0)
    y = jnp.dot(h, w2_ref[0], preferred_element_type=jnp.float32)
    y = y * wg_ref[0, 0][:, None]

    @pl.when(ft == 0)
    def _():
        y_ref[0] = y

    @pl.when(ft != 0)
    def _():
        y_ref[0] += y


def _combine_body(yg_ref, tok_ref, out_ref, sbuf, rbuf, send_sems, recv_sems):
    mx = lax.axis_index("x")
    my = lax.axis_index("y")
    mz = lax.axis_index("z")
    peer = (mx, my, 1 - mz)

    barrier = pltpu.get_barrier_semaphore()
    pl.semaphore_signal(barrier, inc=1, device_id=peer, device_id_type=_MESH)
    pl.semaphore_wait(barrier, 1)

    def rows(base, nrows):
        acc = None
        for e in range(EL):
            tok = tok_ref[e]
            pt = (lax.broadcasted_iota(jnp.int32, (nrows, C), 0) + base
                  == tok[None, :]).astype(jnp.float32)
            c = jnp.dot(pt, yg_ref[e], preferred_element_type=jnp.float32)
            acc = c if acc is None else acc + c
        return acc

    half_t = T // 2
    rdmas = []
    for k in range(2):
        r0 = k * half_t
        sbuf[r0:r0 + half_t] = rows(
            (1 - mz) * T + r0, half_t).astype(jnp.bfloat16)
        rdma = pltpu.make_async_remote_copy(
            src_ref=sbuf.at[pl.ds(r0, half_t)],
            dst_ref=rbuf.at[pl.ds(r0, half_t)],
            send_sem=send_sems.at[k], recv_sem=recv_sems.at[k],
            device_id=peer, device_id_type=_MESH)
        rdma.start()
        rdmas.append(rdma)
    mine = rows(mz * T, T)
    for rdma in rdmas:
        rdma.wait()
    out_ref[...] = mine + rbuf[...].astype(jnp.float32)


def kernel(x, router, W1, W2):
    mz = lax.axis_index("z")

    x_all3, idx_all, w_all = pl.pallas_call(
        _exchange_body,
        out_shape=[
            jax.ShapeDtypeStruct((NZ, T, D), jnp.bfloat16),
            jax.ShapeDtypeStruct((NZ, 2, T), jnp.int32),
            jax.ShapeDtypeStruct((NZ, 2, T), jnp.float32),
        ],
        in_specs=[
            pl.BlockSpec(memory_space=pltpu.VMEM),
            pl.BlockSpec(memory_space=pltpu.VMEM),
        ],
        out_specs=[
            pl.BlockSpec(memory_space=pltpu.VMEM),
            pl.BlockSpec(memory_space=pltpu.VMEM),
            pl.BlockSpec(memory_space=pltpu.VMEM),
        ],
        scratch_shapes=[
            pltpu.VMEM((NZ, D, EL), jnp.float32),
            pltpu.VMEM((T, E), jnp.float32),
            pltpu.SemaphoreType.DMA((4,)),
            pltpu.SemaphoreType.DMA((4,)),
        ],
        compiler_params=pltpu.CompilerParams(collective_id=0),
    )(x, router)

    x_all = x_all3.reshape(NZ * T, D)
    le = idx_all - mz * EL
    le = jnp.where((le >= 0) & (le < EL), le, EL)
    le_f = le.reshape(-1)
    tok_f = jnp.broadcast_to(
        jnp.arange(NZ, dtype=jnp.int32)[:, None, None] * T
        + jnp.arange(T, dtype=jnp.int32)[None, None, :],
        (NZ, 2, T),
    ).reshape(-1)
    w_f = w_all.reshape(-1)

    le_s, tok_s, w_s = lax.sort((le_f, tok_f, w_f), num_keys=1)
    counts = jnp.sum(
        le_f[None, :] == jnp.arange(EL, dtype=jnp.int32)[:, None], axis=1)
    starts = jnp.concatenate(
        [jnp.zeros((1,), counts.dtype), jnp.cumsum(counts)[:-1]])
    tok_pad = jnp.concatenate([tok_s, jnp.zeros((C,), tok_s.dtype)])
    w_pad = jnp.concatenate([w_s, jnp.zeros((C,), w_s.dtype)])
    tok_g = jnp.stack(
        [lax.dynamic_slice(tok_pad, (starts[e],), (C,)) for e in range(EL)])
    w_g = jnp.stack(
        [lax.dynamic_slice(w_pad, (starts[e],), (C,)) for e in range(EL)])
    valid = jnp.arange(C, dtype=jnp.int32)[None, :] < counts[:, None]
    tok_g = jnp.where(valid, tok_g, 0).reshape(EL, 1, C)
    wg = jnp.where(valid, w_g, 0.0).reshape(EL, 1, C)

    yg = pl.pallas_call(
        _ffn_body,
        grid=(EL, F // F_TILE),
        in_specs=[
            pl.BlockSpec((NZ * T, D), lambda e, f: (0, 0)),
            pl.BlockSpec((1, 1, C), lambda e, f: (e, 0, 0)),
            pl.BlockSpec((1, D, F_TILE), lambda e, f: (e, 0, f)),
            pl.BlockSpec((1, F_TILE, D), lambda e, f: (e, f, 0)),
            pl.BlockSpec((1, 1, C), lambda e, f: (e, 0, 0)),
        ],
        out_specs=pl.BlockSpec((1, C, D), lambda e, f: (e, 0, 0)),
        out_shape=jax.ShapeDtypeStruct((EL, C, D), jnp.float32),
        scratch_shapes=[pltpu.VMEM((C, D), jnp.float32)],
        compiler_params=pltpu.CompilerParams(
            dimension_semantics=("arbitrary", "arbitrary")),
    )(x_all, tok_g, W1, W2, wg)

    out = pl.pallas_call(
        _combine_body,
        out_shape=jax.ShapeDtypeStruct((T, D), jnp.float32),
        in_specs=[
            pl.BlockSpec(memory_space=pltpu.VMEM),
            pl.BlockSpec(memory_space=pltpu.VMEM),
        ],
        out_specs=pl.BlockSpec(memory_space=pltpu.VMEM),
        scratch_shapes=[
            pltpu.VMEM((T, D), jnp.bfloat16),
            pltpu.VMEM((T, D), jnp.bfloat16),
            pltpu.SemaphoreType.DMA((2,)),
            pltpu.SemaphoreType.DMA((2,)),
        ],
        compiler_params=pltpu.CompilerParams(collective_id=1),
    )(yg, tok_g.reshape(EL, C))
    return out
